# baseline (speedup 1.0000x reference)
import jax
import jax.numpy as jnp
from jax.experimental import pallas as pl
from jax.experimental.pallas import tpu as pltpu

B, S, D, C = 1024, 200, 128, 49
BB = 32


def _body(s_ref, e_ref, x_ref, w1_ref, b1_ref, w2_ref, b2_ref,
          out1_ref, out2_ref, xmod_ref):
    g = pl.program_id(0)
    iota = jax.lax.broadcasted_iota(jnp.int32, (S, 1), 0)
    for j in range(BB):
        i = g * BB + j
        s = s_ref[i]
        e = e_ref[i]
        xj = x_ref[j]
        vs = x_ref[j, pl.ds(s, 1), :]
        ve = x_ref[j, pl.ds(e, 1), :]
        v = 0.5 * (vs + ve)
        coef = ((iota == s) | (iota == e)).astype(jnp.float32)
        xmod_ref[pl.ds(j * S, S), :] = xj + coef * v
    xmod = xmod_ref[...]
    o1 = jax.nn.sigmoid(
        jnp.dot(xmod, w1_ref[...], preferred_element_type=jnp.float32)
        + b1_ref[...])
    o2 = jax.nn.sigmoid(
        jnp.dot(xmod, w2_ref[...], preferred_element_type=jnp.float32)
        + b2_ref[...])
    out1_ref[...] = o1.reshape(BB, S, C)
    out2_ref[...] = o2.reshape(BB, S, C)


@jax.jit
def kernel(x_lstm, position_s, position_e, W1, b1, W2, b2):
    b1r = b1.reshape(1, C)
    b2r = b2.reshape(1, C)
    pos_s = position_s.astype(jnp.int32)
    pos_e = position_e.astype(jnp.int32)
    grid = B // BB
    out1, out2 = pl.pallas_call(
        _body,
        grid_spec=pltpu.PrefetchScalarGridSpec(
            num_scalar_prefetch=2,
            grid=(grid,),
            in_specs=[
                pl.BlockSpec((BB, S, D), lambda g, *_: (g, 0, 0)),
                pl.BlockSpec((D, C), lambda g, *_: (0, 0)),
                pl.BlockSpec((1, C), lambda g, *_: (0, 0)),
                pl.BlockSpec((D, C), lambda g, *_: (0, 0)),
                pl.BlockSpec((1, C), lambda g, *_: (0, 0)),
            ],
            out_specs=[
                pl.BlockSpec((BB, S, C), lambda g, *_: (g, 0, 0)),
                pl.BlockSpec((BB, S, C), lambda g, *_: (g, 0, 0)),
            ],
            scratch_shapes=[pltpu.VMEM((BB * S, D), jnp.float32)],
        ),
        out_shape=[
            jax.ShapeDtypeStruct((B, S, C), jnp.float32),
            jax.ShapeDtypeStruct((B, S, C), jnp.float32),
        ],
        compiler_params=pltpu.CompilerParams(
            dimension_semantics=("arbitrary",),
        ),
    )(pos_s, pos_e, x_lstm, W1, b1r, W2, b2r)
    return (out1, out2)


# speedup vs baseline: 1.0038x; 1.0038x over previous
import jax
import jax.numpy as jnp
from jax.experimental import pallas as pl
from jax.experimental.pallas import tpu as pltpu

B, S, D, C = 1024, 200, 128, 49
BB = 32


def _body(s_ref, e_ref, x_ref, w1_ref, b1_ref, w2_ref, b2_ref,
          out1_ref, out2_ref, xmod_ref):
    g = pl.program_id(0)
    iota = jax.lax.broadcasted_iota(jnp.int32, (S, 1), 0)
    for j in range(BB):
        i = g * BB + j
        s = s_ref[i]
        e = e_ref[i]
        xj = x_ref[j]
        vs = x_ref[j, pl.ds(s, 1), :]
        ve = x_ref[j, pl.ds(e, 1), :]
        v = 0.5 * (vs + ve)
        coef = ((iota == s) | (iota == e)).astype(jnp.float32)
        xmod_ref[pl.ds(j * S, S), :] = xj + coef * v
    xmod = xmod_ref[...]
    p1 = jnp.dot(xmod, w1_ref[...], preferred_element_type=jnp.float32) \
        + b1_ref[...]
    p2 = jnp.dot(xmod, w2_ref[...], preferred_element_type=jnp.float32) \
        + b2_ref[...]
    o1 = 0.5 * jnp.tanh(0.5 * p1) + 0.5
    o2 = 0.5 * jnp.tanh(0.5 * p2) + 0.5
    out1_ref[...] = o1.reshape(BB, S, C)
    out2_ref[...] = o2.reshape(BB, S, C)


@jax.jit
def kernel(x_lstm, position_s, position_e, W1, b1, W2, b2):
    b1r = b1.reshape(1, C)
    b2r = b2.reshape(1, C)
    pos_s = position_s.astype(jnp.int32)
    pos_e = position_e.astype(jnp.int32)
    grid = B // BB
    out1, out2 = pl.pallas_call(
        _body,
        grid_spec=pltpu.PrefetchScalarGridSpec(
            num_scalar_prefetch=2,
            grid=(grid,),
            in_specs=[
                pl.BlockSpec((BB, S, D), lambda g, *_: (g, 0, 0)),
                pl.BlockSpec((D, C), lambda g, *_: (0, 0)),
                pl.BlockSpec((1, C), lambda g, *_: (0, 0)),
                pl.BlockSpec((D, C), lambda g, *_: (0, 0)),
                pl.BlockSpec((1, C), lambda g, *_: (0, 0)),
            ],
            out_specs=[
                pl.BlockSpec((BB, S, C), lambda g, *_: (g, 0, 0)),
                pl.BlockSpec((BB, S, C), lambda g, *_: (g, 0, 0)),
            ],
            scratch_shapes=[pltpu.VMEM((BB * S, D), jnp.float32)],
        ),
        out_shape=[
            jax.ShapeDtypeStruct((B, S, C), jnp.float32),
            jax.ShapeDtypeStruct((B, S, C), jnp.float32),
        ],
        compiler_params=pltpu.CompilerParams(
            dimension_semantics=("parallel",),
        ),
    )(pos_s, pos_e, x_lstm, W1, b1r, W2, b2r)
    return (out1, out2)
